# baseline (speedup 1.0000x reference)
import functools
import jax
import jax.numpy as jnp
from jax import lax
from jax.experimental import pallas as pl
from jax.experimental.pallas import tpu as pltpu
from jax.experimental.pallas import tpu_sc as plsc

_B = 16384
_D = 64
_H = 128

_info = plsc.get_sparse_core_info()
_NC, _NS = _info.num_cores, _info.num_subcores
_NW = _NC * _NS
_BPW = _B // _NW


_sc_mesh = plsc.VectorSubcoreMesh(core_axis_name="c", subcore_axis_name="s")


@functools.partial(
    pl.kernel,
    out_type=(
        jax.ShapeDtypeStruct((_B, _D), jnp.float32),
        jax.ShapeDtypeStruct((_B, _D), jnp.float32),
    ),
    mesh=_sc_mesh,
    scratch_types=[
        pltpu.VMEM((_BPW,), jnp.int32),
        pltpu.VMEM((_BPW,), jnp.int32),
        pltpu.VMEM((_BPW, _D), jnp.float32),
        pltpu.VMEM((_BPW, _D), jnp.float32),
        pltpu.SemaphoreType.DMA,
        pltpu.SemaphoreType.DMA,
    ],
    compiler_params=pltpu.CompilerParams(use_tc_tiling_on_sc=False),
)
def _sc_gather(user_hbm, art_hbm, utab_hbm, atab_hbm, ue_hbm, ae_hbm,
               idx_u, idx_a, rows_u, rows_a, sem_u, sem_a):
    wid = lax.axis_index("s") * _NC + lax.axis_index("c")
    base = wid * _BPW
    pltpu.sync_copy(user_hbm.at[pl.ds(base, _BPW)], idx_u)
    pltpu.sync_copy(art_hbm.at[pl.ds(base, _BPW)], idx_a)
    cu = pltpu.async_copy(utab_hbm.at[idx_u], rows_u, sem_u)
    ca = pltpu.async_copy(atab_hbm.at[idx_a], rows_a, sem_a)
    cu.wait()
    pltpu.sync_copy(rows_u, ue_hbm.at[pl.ds(base, _BPW)])
    ca.wait()
    pltpu.sync_copy(rows_a, ae_hbm.at[pl.ds(base, _BPW)])


_BLK = 2048


def _mlp_body(ue_ref, ae_ref, w1a_ref, w1b_ref, b1_ref, w2_ref, b2_ref, out_ref):
    h = jnp.dot(ue_ref[...], w1a_ref[...], preferred_element_type=jnp.float32)
    h += jnp.dot(ae_ref[...], w1b_ref[...], preferred_element_type=jnp.float32)
    h = jnp.maximum(h + b1_ref[...], 0.0)
    o = jnp.dot(h, w2_ref[...], preferred_element_type=jnp.float32)
    out_ref[...] = jax.nn.sigmoid(o + b2_ref[...])


_mlp = pl.pallas_call(
    _mlp_body,
    grid=(_B // _BLK,),
    in_specs=[
        pl.BlockSpec((_BLK, _D), lambda i: (i, 0)),
        pl.BlockSpec((_BLK, _D), lambda i: (i, 0)),
        pl.BlockSpec((_D, _H), lambda i: (0, 0)),
        pl.BlockSpec((_D, _H), lambda i: (0, 0)),
        pl.BlockSpec((1, _H), lambda i: (0, 0)),
        pl.BlockSpec((_H, 1), lambda i: (0, 0)),
        pl.BlockSpec((1, 1), lambda i: (0, 0)),
    ],
    out_specs=pl.BlockSpec((_BLK, 1), lambda i: (i, 0)),
    out_shape=jax.ShapeDtypeStruct((_B, 1), jnp.float32),
)


@jax.jit
def kernel(user, artwork, user_table, artwork_table, W1, b1, W2, b2):
    ue, ae = _sc_gather(user, artwork, user_table, artwork_table)
    w1a = W1[:, :_D].T
    w1b = W1[:, _D:].T
    return _mlp(ue, ae, w1a, w1b, b1.reshape(1, _H), W2.T, b2.reshape(1, 1))
